# baseline (speedup 1.0000x reference)
import jax
import jax.numpy as jnp
from jax import lax
from jax.experimental import pallas as pl
from jax.experimental.pallas import tpu as pltpu
from jax.experimental.pallas import tpu_sc as plsc

N_NODES = 10000
N_EDGES = 320000
D_IN = 128
D_OUT = 64

NC = 2
NS = 16
NW = NC * NS
CHUNK = 80
ROWS_PER_TILE = N_EDGES // (NW * CHUNK)
N_PAD = 10240
NPT = N_PAD // NS

_mesh = plsc.VectorSubcoreMesh(
    core_axis_name="c", subcore_axis_name="s", num_cores=NC, num_subcores=NS)


def _deg_body(dst3d_hbm, out_hbm, dst_idx, ones_v, zero_v, shared_deg, sem):
    c = lax.axis_index("c")
    s = lax.axis_index("s")

    def fill(i, _):
        ones_v[pl.ds(i * 16, 16)] = jnp.ones((16,), jnp.float32)
        return 0
    lax.fori_loop(0, CHUNK // 16, fill, 0)

    def zfill(i, _):
        zero_v[pl.ds(i * 16, 16)] = jnp.zeros((16,), jnp.float32)
        return 0
    lax.fori_loop(0, NPT // 16, zfill, 0)

    pltpu.sync_copy(zero_v, shared_deg.at[pl.ds(s * NPT, NPT)])
    plsc.subcore_barrier()

    pltpu.sync_copy(dst3d_hbm.at[c * NS + s], dst_idx)

    def body(j, _):
        pltpu.sync_copy(ones_v, shared_deg.at[dst_idx.at[j]], add=True)
        return 0
    lax.fori_loop(0, ROWS_PER_TILE, body, 0)

    plsc.subcore_barrier()
    pltpu.sync_copy(shared_deg.at[pl.ds(s * NPT, NPT)],
                    out_hbm.at[pl.ds(c * N_PAD + s * NPT, NPT)])


_deg_kernel = pl.kernel(
    _deg_body,
    out_type=jax.ShapeDtypeStruct((NC * N_PAD,), jnp.float32),
    mesh=_mesh,
    scratch_types=[
        pltpu.VMEM((ROWS_PER_TILE, CHUNK), jnp.int32),
        pltpu.VMEM((CHUNK,), jnp.float32),
        pltpu.VMEM((NPT,), jnp.float32),
        pltpu.VMEM_SHARED((N_PAD,), jnp.float32),
        pltpu.SemaphoreType.DMA,
    ],
)


def _hop_body(h_hbm, src3d_hbm, dst3d_hbm, out_hbm,
              src_idx, dst_idx, rows_a, rows_b, zrows_v, shared_g,
              sem_a, sem_b, sem_sa, sem_sb):
    c = lax.axis_index("c")
    s = lax.axis_index("s")

    zchunk = NPT // 5

    def zfill(k, _):
        zrows_v[k // 4, pl.ds((k % 4) * 16, 16)] = jnp.zeros((16,), jnp.float32)
        return 0
    lax.fori_loop(0, zchunk * 4, zfill, 0)

    pltpu.sync_copy(src3d_hbm.at[c * NS + s], src_idx)
    pltpu.sync_copy(dst3d_hbm.at[c * NS + s], dst_idx)

    def zcopy(k, _):
        pltpu.sync_copy(zrows_v, shared_g.at[pl.ds(s * NPT + k * zchunk, zchunk)])
        return 0
    lax.fori_loop(0, 5, zcopy, 0)
    plsc.subcore_barrier()

    def gather(j, buf, sem):
        pltpu.async_copy(h_hbm.at[src_idx.at[j]], buf, sem)

    def gwait(buf, sem):
        pltpu.make_async_copy(h_hbm.at[src_idx.at[0]], buf, sem).wait()

    def scat(j, buf, sem):
        pltpu.async_copy(buf, shared_g.at[dst_idx.at[j]], sem, add=True)

    def swait(buf, sem):
        pltpu.make_async_copy(buf, shared_g.at[dst_idx.at[0]], sem).wait()

    gather(0, rows_a, sem_a)
    gather(1, rows_b, sem_b)

    def body(k, _):
        j = 2 * k
        gwait(rows_a, sem_a)
        scat(j, rows_a, sem_sa)
        gwait(rows_b, sem_b)
        scat(j + 1, rows_b, sem_sb)
        swait(rows_a, sem_sa)
        gather(j + 2, rows_a, sem_a)
        swait(rows_b, sem_sb)

        @pl.when(j + 3 < ROWS_PER_TILE)
        def _():
            gather(j + 3, rows_b, sem_b)
        return 0
    lax.fori_loop(0, (ROWS_PER_TILE - 1) // 2, body, 0)
    gwait(rows_a, sem_a)
    scat(ROWS_PER_TILE - 1, rows_a, sem_sa)
    swait(rows_a, sem_sa)

    plsc.subcore_barrier()
    pltpu.sync_copy(shared_g.at[pl.ds(s * NPT, NPT)],
                    out_hbm.at[c, pl.ds(s * NPT, NPT)])


_hop_kernel = pl.kernel(
    _hop_body,
    out_type=jax.ShapeDtypeStruct((NC, N_PAD, D_OUT), jnp.float32),
    mesh=_mesh,
    compiler_params=pltpu.CompilerParams(use_tc_tiling_on_sc=False),
    scratch_types=[
        pltpu.VMEM((ROWS_PER_TILE, CHUNK), jnp.int32),
        pltpu.VMEM((ROWS_PER_TILE, CHUNK), jnp.int32),
        pltpu.VMEM((CHUNK, D_OUT), jnp.float32),
        pltpu.VMEM((CHUNK, D_OUT), jnp.float32),
        pltpu.VMEM((N_PAD // NS // 5, D_OUT), jnp.float32),
        pltpu.VMEM_SHARED((N_PAD, D_OUT), jnp.float32),
        pltpu.SemaphoreType.DMA,
        pltpu.SemaphoreType.DMA,
        pltpu.SemaphoreType.DMA,
        pltpu.SemaphoreType.DMA,
    ],
)


ROW_BLK = 1000


def _prep_body(x_ref, w_ref, da_ref, db_ref, h0_ref, nrm_ref, inv_ref):
    deg = jnp.maximum(da_ref[...] + db_ref[...], 1.0)
    nrm = lax.rsqrt(deg)
    h0_ref[...] = jnp.dot(x_ref[...], w_ref[...],
                          preferred_element_type=jnp.float32) * nrm
    nrm_ref[...] = nrm
    inv_ref[...] = 1.0 / deg


def _mid_body(ga_ref, gb_ref, inv_ref, h1_ref):
    h1_ref[...] = (ga_ref[...] + gb_ref[...]) * inv_ref[...]


def _final_body(ga_ref, gb_ref, nrm_ref, b_ref, out_ref):
    out_ref[...] = (ga_ref[...] + gb_ref[...]) * nrm_ref[...] + b_ref[...]


def _row_spec(cols):
    return pl.BlockSpec((ROW_BLK, cols), lambda i: (i, 0))


def kernel(in_feat, edge_index, W, b):
    src3d = edge_index[0].astype(jnp.int32).reshape(NW, ROWS_PER_TILE, CHUNK)
    dst3d = edge_index[1].astype(jnp.int32).reshape(NW, ROWS_PER_TILE, CHUNK)

    deg_part = _deg_kernel(dst3d).reshape(NC, N_PAD)
    da = deg_part[0, :N_NODES].reshape(N_NODES, 1)
    db = deg_part[1, :N_NODES].reshape(N_NODES, 1)

    grid = (N_NODES // ROW_BLK,)
    h0, nrm, inv = pl.pallas_call(
        _prep_body,
        grid=grid,
        in_specs=[
            _row_spec(D_IN),
            pl.BlockSpec((D_IN, D_OUT), lambda i: (0, 0)),
            _row_spec(1),
            _row_spec(1),
        ],
        out_specs=[_row_spec(D_OUT), _row_spec(1), _row_spec(1)],
        out_shape=[
            jax.ShapeDtypeStruct((N_NODES, D_OUT), jnp.float32),
            jax.ShapeDtypeStruct((N_NODES, 1), jnp.float32),
            jax.ShapeDtypeStruct((N_NODES, 1), jnp.float32),
        ],
    )(in_feat, W, da, db)

    g1 = _hop_kernel(h0, src3d, dst3d)

    h1 = pl.pallas_call(
        _mid_body,
        grid=grid,
        in_specs=[_row_spec(D_OUT), _row_spec(D_OUT), _row_spec(1)],
        out_specs=_row_spec(D_OUT),
        out_shape=jax.ShapeDtypeStruct((N_NODES, D_OUT), jnp.float32),
    )(g1[0, :N_NODES], g1[1, :N_NODES], inv)

    g2 = _hop_kernel(h1, src3d, dst3d)

    out = pl.pallas_call(
        _final_body,
        grid=grid,
        in_specs=[
            _row_spec(D_OUT),
            _row_spec(D_OUT),
            _row_spec(1),
            pl.BlockSpec((1, D_OUT), lambda i: (0, 0)),
        ],
        out_specs=_row_spec(D_OUT),
        out_shape=jax.ShapeDtypeStruct((N_NODES, D_OUT), jnp.float32),
    )(g2[0, :N_NODES], g2[1, :N_NODES], nrm, b.reshape(1, D_OUT))

    return out


# speedup vs baseline: 1.0942x; 1.0942x over previous
import jax
import jax.numpy as jnp
from jax import lax
from jax.experimental import pallas as pl
from jax.experimental.pallas import tpu as pltpu
from jax.experimental.pallas import tpu_sc as plsc

N_NODES = 10000
N_EDGES = 320000
D_IN = 128
D_OUT = 64

NC = 2
NS = 16
NW = NC * NS
CHUNK = 80
ROWS_PER_TILE = N_EDGES // (NW * CHUNK)
N_PAD = 10240
NPT = N_PAD // NS

_mesh = plsc.VectorSubcoreMesh(
    core_axis_name="c", subcore_axis_name="s", num_cores=NC, num_subcores=NS)


def _deg_body(dst3d_hbm, out_hbm, dst_idx, ones_v, zero_v, shared_deg, sem):
    c = lax.axis_index("c")
    s = lax.axis_index("s")

    def fill(i, _):
        ones_v[pl.ds(i * 16, 16)] = jnp.ones((16,), jnp.float32)
        return 0
    lax.fori_loop(0, CHUNK // 16, fill, 0)

    def zfill(i, _):
        zero_v[pl.ds(i * 16, 16)] = jnp.zeros((16,), jnp.float32)
        return 0
    lax.fori_loop(0, NPT // 16, zfill, 0)

    pltpu.sync_copy(zero_v, shared_deg.at[pl.ds(s * NPT, NPT)])
    plsc.subcore_barrier()

    pltpu.sync_copy(dst3d_hbm.at[c * NS + s], dst_idx)

    def body(j, _):
        pltpu.sync_copy(ones_v, shared_deg.at[dst_idx.at[j]], add=True)
        return 0
    lax.fori_loop(0, ROWS_PER_TILE, body, 0)

    plsc.subcore_barrier()
    pltpu.sync_copy(shared_deg.at[pl.ds(s * NPT, NPT)],
                    out_hbm.at[pl.ds(c * N_PAD + s * NPT, NPT)])


_deg_kernel = pl.kernel(
    _deg_body,
    out_type=jax.ShapeDtypeStruct((NC * N_PAD,), jnp.float32),
    mesh=_mesh,
    scratch_types=[
        pltpu.VMEM((ROWS_PER_TILE, CHUNK), jnp.int32),
        pltpu.VMEM((CHUNK,), jnp.float32),
        pltpu.VMEM((NPT,), jnp.float32),
        pltpu.VMEM_SHARED((N_PAD,), jnp.float32),
        pltpu.SemaphoreType.DMA,
    ],
)


def _hop_body(h_hbm, src3d_hbm, dst3d_hbm, out_hbm,
              src_idx, dst_idx, rows_a, rows_b, zrows_v, shared_g,
              sem_a, sem_b):
    c = lax.axis_index("c")
    s = lax.axis_index("s")

    zchunk = NPT // 5

    def zfill(k, _):
        zrows_v[k // 4, pl.ds((k % 4) * 16, 16)] = jnp.zeros((16,), jnp.float32)
        return 0
    lax.fori_loop(0, zchunk * 4, zfill, 0)

    pltpu.sync_copy(src3d_hbm.at[c * NS + s], src_idx)
    pltpu.sync_copy(dst3d_hbm.at[c * NS + s], dst_idx)

    def zcopy(k, _):
        pltpu.sync_copy(zrows_v, shared_g.at[pl.ds(s * NPT + k * zchunk, zchunk)])
        return 0
    lax.fori_loop(0, 5, zcopy, 0)
    plsc.subcore_barrier()

    def gather(j, buf, sem):
        pltpu.async_copy(h_hbm.at[src_idx.at[j]], buf, sem)

    def gwait(buf, sem):
        pltpu.make_async_copy(h_hbm.at[src_idx.at[0]], buf, sem).wait()

    def scat(j, buf):
        pltpu.sync_copy(buf, shared_g.at[dst_idx.at[j]], add=True)

    gather(0, rows_a, sem_a)

    def body(k, _):
        gather(2 * k + 1, rows_b, sem_b)
        gwait(rows_a, sem_a)
        scat(2 * k, rows_a)
        gather(2 * k + 2, rows_a, sem_a)
        gwait(rows_b, sem_b)
        scat(2 * k + 1, rows_b)
        return 0
    lax.fori_loop(0, (ROWS_PER_TILE - 1) // 2, body, 0)
    gwait(rows_a, sem_a)
    scat(ROWS_PER_TILE - 1, rows_a)

    plsc.subcore_barrier()
    pltpu.sync_copy(shared_g.at[pl.ds(s * NPT, NPT)],
                    out_hbm.at[c, pl.ds(s * NPT, NPT)])


_hop_kernel = pl.kernel(
    _hop_body,
    out_type=jax.ShapeDtypeStruct((NC, N_PAD, D_OUT), jnp.float32),
    mesh=_mesh,
    compiler_params=pltpu.CompilerParams(use_tc_tiling_on_sc=False),
    scratch_types=[
        pltpu.VMEM((ROWS_PER_TILE, CHUNK), jnp.int32),
        pltpu.VMEM((ROWS_PER_TILE, CHUNK), jnp.int32),
        pltpu.VMEM((CHUNK, D_OUT), jnp.float32),
        pltpu.VMEM((CHUNK, D_OUT), jnp.float32),
        pltpu.VMEM((N_PAD // NS // 5, D_OUT), jnp.float32),
        pltpu.VMEM_SHARED((N_PAD, D_OUT), jnp.float32),
        pltpu.SemaphoreType.DMA,
        pltpu.SemaphoreType.DMA,
    ],
)


ROW_BLK = 1000


def _prep_body(x_ref, w_ref, da_ref, db_ref, h0_ref, nrm_ref, inv_ref):
    deg = jnp.maximum(da_ref[...] + db_ref[...], 1.0)
    nrm = lax.rsqrt(deg)
    h0_ref[...] = jnp.dot(x_ref[...], w_ref[...],
                          preferred_element_type=jnp.float32) * nrm
    nrm_ref[...] = nrm
    inv_ref[...] = 1.0 / deg


def _mid_body(ga_ref, gb_ref, inv_ref, h1_ref):
    h1_ref[...] = (ga_ref[...] + gb_ref[...]) * inv_ref[...]


def _final_body(ga_ref, gb_ref, nrm_ref, b_ref, out_ref):
    out_ref[...] = (ga_ref[...] + gb_ref[...]) * nrm_ref[...] + b_ref[...]


def _row_spec(cols):
    return pl.BlockSpec((ROW_BLK, cols), lambda i: (i, 0))


def kernel(in_feat, edge_index, W, b):
    src3d = edge_index[0].astype(jnp.int32).reshape(NW, ROWS_PER_TILE, CHUNK)
    dst3d = edge_index[1].astype(jnp.int32).reshape(NW, ROWS_PER_TILE, CHUNK)

    deg_part = _deg_kernel(dst3d).reshape(NC, N_PAD)
    da = deg_part[0, :N_NODES].reshape(N_NODES, 1)
    db = deg_part[1, :N_NODES].reshape(N_NODES, 1)

    grid = (N_NODES // ROW_BLK,)
    h0, nrm, inv = pl.pallas_call(
        _prep_body,
        grid=grid,
        in_specs=[
            _row_spec(D_IN),
            pl.BlockSpec((D_IN, D_OUT), lambda i: (0, 0)),
            _row_spec(1),
            _row_spec(1),
        ],
        out_specs=[_row_spec(D_OUT), _row_spec(1), _row_spec(1)],
        out_shape=[
            jax.ShapeDtypeStruct((N_NODES, D_OUT), jnp.float32),
            jax.ShapeDtypeStruct((N_NODES, 1), jnp.float32),
            jax.ShapeDtypeStruct((N_NODES, 1), jnp.float32),
        ],
    )(in_feat, W, da, db)

    g1 = _hop_kernel(h0, src3d, dst3d)

    h1 = pl.pallas_call(
        _mid_body,
        grid=grid,
        in_specs=[_row_spec(D_OUT), _row_spec(D_OUT), _row_spec(1)],
        out_specs=_row_spec(D_OUT),
        out_shape=jax.ShapeDtypeStruct((N_NODES, D_OUT), jnp.float32),
    )(g1[0, :N_NODES], g1[1, :N_NODES], inv)

    g2 = _hop_kernel(h1, src3d, dst3d)

    out = pl.pallas_call(
        _final_body,
        grid=grid,
        in_specs=[
            _row_spec(D_OUT),
            _row_spec(D_OUT),
            _row_spec(1),
            pl.BlockSpec((1, D_OUT), lambda i: (0, 0)),
        ],
        out_specs=_row_spec(D_OUT),
        out_shape=jax.ShapeDtypeStruct((N_NODES, D_OUT), jnp.float32),
    )(g2[0, :N_NODES], g2[1, :N_NODES], nrm, b.reshape(1, D_OUT))

    return out


# speedup vs baseline: 1.1496x; 1.0507x over previous
import jax
import jax.numpy as jnp
from jax import lax
from jax.experimental import pallas as pl
from jax.experimental.pallas import tpu as pltpu
from jax.experimental.pallas import tpu_sc as plsc

N_NODES = 10000
N_EDGES = 320000
D_IN = 128
D_OUT = 64

NC = 2
NS = 16
NW = NC * NS
CHUNK = 80
ROWS_PER_TILE = N_EDGES // (NW * CHUNK)
N_PAD = 10240
NPT = N_PAD // NS

_mesh = plsc.VectorSubcoreMesh(
    core_axis_name="c", subcore_axis_name="s", num_cores=NC, num_subcores=NS)


def _deg_body(dst3d_hbm, out_hbm, dst_idx, ones_v, zero_v, shared_deg, sem,
              ssem):
    c = lax.axis_index("c")
    s = lax.axis_index("s")

    def fill(i, _):
        ones_v[pl.ds(i * 16, 16)] = jnp.ones((16,), jnp.float32)
        return 0
    lax.fori_loop(0, CHUNK // 16, fill, 0)

    def zfill(i, _):
        zero_v[pl.ds(i * 16, 16)] = jnp.zeros((16,), jnp.float32)
        return 0
    lax.fori_loop(0, NPT // 16, zfill, 0)

    pltpu.sync_copy(zero_v, shared_deg.at[pl.ds(s * NPT, NPT)])
    plsc.subcore_barrier()

    pltpu.sync_copy(dst3d_hbm.at[c * NS + s], dst_idx)

    def body(j, _):
        pltpu.async_copy(ones_v, shared_deg.at[dst_idx.at[j]], ssem, add=True)
        return 0
    lax.fori_loop(0, ROWS_PER_TILE, body, 0)

    def drain(j, _):
        pltpu.make_async_copy(ones_v, shared_deg.at[dst_idx.at[0]], ssem).wait()
        return 0
    lax.fori_loop(0, ROWS_PER_TILE, drain, 0)

    plsc.subcore_barrier()
    pltpu.sync_copy(shared_deg.at[pl.ds(s * NPT, NPT)],
                    out_hbm.at[pl.ds(c * N_PAD + s * NPT, NPT)])


_deg_kernel = pl.kernel(
    _deg_body,
    out_type=jax.ShapeDtypeStruct((NC * N_PAD,), jnp.float32),
    mesh=_mesh,
    scratch_types=[
        pltpu.VMEM((ROWS_PER_TILE, CHUNK), jnp.int32),
        pltpu.VMEM((CHUNK,), jnp.float32),
        pltpu.VMEM((NPT,), jnp.float32),
        pltpu.VMEM_SHARED((N_PAD,), jnp.float32),
        pltpu.SemaphoreType.DMA,
        pltpu.SemaphoreType.DMA,
    ],
)


def _hop_body(h_hbm, src3d_hbm, dst3d_hbm, out_hbm,
              src_idx, dst_idx, rows_a, rows_b, zrows_v, shared_g,
              sem_a, sem_b):
    c = lax.axis_index("c")
    s = lax.axis_index("s")

    zchunk = NPT // 5

    def zfill(k, _):
        zrows_v[k // 4, pl.ds((k % 4) * 16, 16)] = jnp.zeros((16,), jnp.float32)
        return 0
    lax.fori_loop(0, zchunk * 4, zfill, 0)

    pltpu.sync_copy(src3d_hbm.at[c * NS + s], src_idx)
    pltpu.sync_copy(dst3d_hbm.at[c * NS + s], dst_idx)

    def zcopy(k, _):
        pltpu.sync_copy(zrows_v, shared_g.at[pl.ds(s * NPT + k * zchunk, zchunk)])
        return 0
    lax.fori_loop(0, 5, zcopy, 0)
    plsc.subcore_barrier()

    def gather(j, buf, sem):
        pltpu.async_copy(h_hbm.at[src_idx.at[j]], buf, sem)

    def gwait(buf, sem):
        pltpu.make_async_copy(h_hbm.at[src_idx.at[0]], buf, sem).wait()

    def scat(j, buf):
        pltpu.sync_copy(buf, shared_g.at[dst_idx.at[j]], add=True)

    gather(0, rows_a, sem_a)

    def body(k, _):
        gather(2 * k + 1, rows_b, sem_b)
        gwait(rows_a, sem_a)
        scat(2 * k, rows_a)
        gather(2 * k + 2, rows_a, sem_a)
        gwait(rows_b, sem_b)
        scat(2 * k + 1, rows_b)
        return 0
    lax.fori_loop(0, (ROWS_PER_TILE - 1) // 2, body, 0)
    gwait(rows_a, sem_a)
    scat(ROWS_PER_TILE - 1, rows_a)

    plsc.subcore_barrier()
    pltpu.sync_copy(shared_g.at[pl.ds(s * NPT, NPT)],
                    out_hbm.at[c, pl.ds(s * NPT, NPT)])


_hop_kernel = pl.kernel(
    _hop_body,
    out_type=jax.ShapeDtypeStruct((NC, N_PAD, D_OUT), jnp.float32),
    mesh=_mesh,
    compiler_params=pltpu.CompilerParams(use_tc_tiling_on_sc=False),
    scratch_types=[
        pltpu.VMEM((ROWS_PER_TILE, CHUNK), jnp.int32),
        pltpu.VMEM((ROWS_PER_TILE, CHUNK), jnp.int32),
        pltpu.VMEM((CHUNK, D_OUT), jnp.float32),
        pltpu.VMEM((CHUNK, D_OUT), jnp.float32),
        pltpu.VMEM((N_PAD // NS // 5, D_OUT), jnp.float32),
        pltpu.VMEM_SHARED((N_PAD, D_OUT), jnp.float32),
        pltpu.SemaphoreType.DMA,
        pltpu.SemaphoreType.DMA,
    ],
)


ROW_BLK = 10000


def _prep_body(x_ref, w_ref, da_ref, db_ref, h0_ref, nrm_ref, inv_ref):
    deg = jnp.maximum(da_ref[...] + db_ref[...], 1.0)
    nrm = lax.rsqrt(deg)
    h0_ref[...] = jnp.dot(x_ref[...], w_ref[...],
                          preferred_element_type=jnp.float32) * nrm
    nrm_ref[...] = nrm
    inv_ref[...] = 1.0 / deg


def _mid_body(ga_ref, gb_ref, inv_ref, h1_ref):
    h1_ref[...] = (ga_ref[...] + gb_ref[...]) * inv_ref[...]


def _final_body(ga_ref, gb_ref, nrm_ref, b_ref, out_ref):
    out_ref[...] = (ga_ref[...] + gb_ref[...]) * nrm_ref[...] + b_ref[...]


def _row_spec(cols):
    return pl.BlockSpec((ROW_BLK, cols), lambda i: (i, 0))


def kernel(in_feat, edge_index, W, b):
    src3d = edge_index[0].astype(jnp.int32).reshape(NW, ROWS_PER_TILE, CHUNK)
    dst3d = edge_index[1].astype(jnp.int32).reshape(NW, ROWS_PER_TILE, CHUNK)

    deg_part = _deg_kernel(dst3d).reshape(NC, N_PAD)
    da = deg_part[0, :N_NODES].reshape(N_NODES, 1)
    db = deg_part[1, :N_NODES].reshape(N_NODES, 1)

    grid = (N_NODES // ROW_BLK,)
    h0, nrm, inv = pl.pallas_call(
        _prep_body,
        grid=grid,
        in_specs=[
            _row_spec(D_IN),
            pl.BlockSpec((D_IN, D_OUT), lambda i: (0, 0)),
            _row_spec(1),
            _row_spec(1),
        ],
        out_specs=[_row_spec(D_OUT), _row_spec(1), _row_spec(1)],
        out_shape=[
            jax.ShapeDtypeStruct((N_NODES, D_OUT), jnp.float32),
            jax.ShapeDtypeStruct((N_NODES, 1), jnp.float32),
            jax.ShapeDtypeStruct((N_NODES, 1), jnp.float32),
        ],
    )(in_feat, W, da, db)

    g1 = _hop_kernel(h0, src3d, dst3d)

    h1 = pl.pallas_call(
        _mid_body,
        grid=grid,
        in_specs=[_row_spec(D_OUT), _row_spec(D_OUT), _row_spec(1)],
        out_specs=_row_spec(D_OUT),
        out_shape=jax.ShapeDtypeStruct((N_NODES, D_OUT), jnp.float32),
    )(g1[0, :N_NODES], g1[1, :N_NODES], inv)

    g2 = _hop_kernel(h1, src3d, dst3d)

    out = pl.pallas_call(
        _final_body,
        grid=grid,
        in_specs=[
            _row_spec(D_OUT),
            _row_spec(D_OUT),
            _row_spec(1),
            pl.BlockSpec((1, D_OUT), lambda i: (0, 0)),
        ],
        out_specs=_row_spec(D_OUT),
        out_shape=jax.ShapeDtypeStruct((N_NODES, D_OUT), jnp.float32),
    )(g2[0, :N_NODES], g2[1, :N_NODES], nrm, b.reshape(1, D_OUT))

    return out


# speedup vs baseline: 1.3903x; 1.2094x over previous
import jax
import jax.numpy as jnp
from jax import lax
from jax.experimental import pallas as pl
from jax.experimental.pallas import tpu as pltpu
from jax.experimental.pallas import tpu_sc as plsc

N_NODES = 10000
N_EDGES = 320000
D_IN = 128
D_OUT = 64

NC = 2
NS = 16
NW = NC * NS
CHUNK = 80
ROWS_PER_TILE = N_EDGES // (NW * CHUNK)
N_PAD = 10240
NPT = N_PAD // NS

_mesh = plsc.VectorSubcoreMesh(
    core_axis_name="c", subcore_axis_name="s", num_cores=NC, num_subcores=NS)


def _deg_body(dst3d_hbm, out_hbm, dst_idx, ones_v, zero_v, shared_deg, sem,
              ssem):
    c = lax.axis_index("c")
    s = lax.axis_index("s")

    def fill(i, _):
        ones_v[pl.ds(i * 16, 16)] = jnp.ones((16,), jnp.float32)
        return 0
    lax.fori_loop(0, CHUNK // 16, fill, 0)

    def zfill(i, _):
        zero_v[pl.ds(i * 16, 16)] = jnp.zeros((16,), jnp.float32)
        return 0
    lax.fori_loop(0, NPT // 16, zfill, 0)

    pltpu.sync_copy(zero_v, shared_deg.at[pl.ds(s * NPT, NPT)])
    plsc.subcore_barrier()

    pltpu.sync_copy(dst3d_hbm.at[c * NS + s], dst_idx)

    def body(j, _):
        pltpu.async_copy(ones_v, shared_deg.at[dst_idx.at[j]], ssem, add=True)
        return 0
    lax.fori_loop(0, ROWS_PER_TILE, body, 0)

    def drain(j, _):
        pltpu.make_async_copy(ones_v, shared_deg.at[dst_idx.at[0]], ssem).wait()
        return 0
    lax.fori_loop(0, ROWS_PER_TILE, drain, 0)

    plsc.subcore_barrier()
    pltpu.sync_copy(shared_deg.at[pl.ds(s * NPT, NPT)],
                    out_hbm.at[pl.ds(c * N_PAD + s * NPT, NPT)])


_deg_kernel = pl.kernel(
    _deg_body,
    out_type=jax.ShapeDtypeStruct((NC * N_PAD,), jnp.float32),
    mesh=_mesh,
    scratch_types=[
        pltpu.VMEM((ROWS_PER_TILE, CHUNK), jnp.int32),
        pltpu.VMEM((CHUNK,), jnp.float32),
        pltpu.VMEM((NPT,), jnp.float32),
        pltpu.VMEM_SHARED((N_PAD,), jnp.float32),
        pltpu.SemaphoreType.DMA,
        pltpu.SemaphoreType.DMA,
    ],
)


GD = 5


def _hop_body(h_hbm, src3d_hbm, dst3d_hbm, out_hbm,
              src_idx, dst_idx, *rest):
    bufs = rest[:2 * GD]
    zrows_v = rest[2 * GD]
    shared_g = rest[2 * GD + 1]
    sems = rest[2 * GD + 2:]
    c = lax.axis_index("c")
    s = lax.axis_index("s")

    zchunk = NPT // 5

    def zfill(k, _):
        zrows_v[k // 4, pl.ds((k % 4) * 16, 16)] = jnp.zeros((16,), jnp.float32)
        return 0
    lax.fori_loop(0, zchunk * 4, zfill, 0)

    pltpu.sync_copy(src3d_hbm.at[c * NS + s], src_idx)
    pltpu.sync_copy(dst3d_hbm.at[c * NS + s], dst_idx)

    def zcopy(k, _):
        pltpu.sync_copy(zrows_v, shared_g.at[pl.ds(s * NPT + k * zchunk, zchunk)])
        return 0
    lax.fori_loop(0, 5, zcopy, 0)
    plsc.subcore_barrier()

    def gather(j, buf, sem):
        pltpu.async_copy(h_hbm.at[src_idx.at[j]], buf, sem)

    def gwait(buf, sem):
        pltpu.make_async_copy(h_hbm.at[src_idx.at[0]], buf, sem).wait()

    def scat(j, buf):
        pltpu.sync_copy(buf, shared_g.at[dst_idx.at[j]], add=True)

    for i in range(GD):
        gather(i, bufs[i], sems[i])

    def body(m, _):
        def run(cur, nxt):
            for i in range(GD):
                jn = (m + 1) * GD + i

                @pl.when(jn < ROWS_PER_TILE)
                def _():
                    gather(jn, bufs[nxt + i], sems[nxt + i])
            for i in range(GD):
                gwait(bufs[cur + i], sems[cur + i])
                scat(m * GD + i, bufs[cur + i])

        @pl.when(m % 2 == 0)
        def _():
            run(0, GD)

        @pl.when(m % 2 == 1)
        def _():
            run(GD, 0)
        return 0
    lax.fori_loop(0, ROWS_PER_TILE // GD, body, 0)

    plsc.subcore_barrier()
    pltpu.sync_copy(shared_g.at[pl.ds(s * NPT, NPT)],
                    out_hbm.at[c, pl.ds(s * NPT, NPT)])


_hop_kernel = pl.kernel(
    _hop_body,
    out_type=jax.ShapeDtypeStruct((NC, N_PAD, D_OUT), jnp.float32),
    mesh=_mesh,
    compiler_params=pltpu.CompilerParams(use_tc_tiling_on_sc=False),
    scratch_types=(
        [
            pltpu.VMEM((ROWS_PER_TILE, CHUNK), jnp.int32),
            pltpu.VMEM((ROWS_PER_TILE, CHUNK), jnp.int32),
        ]
        + [pltpu.VMEM((CHUNK, D_OUT), jnp.float32)] * (2 * GD)
        + [
            pltpu.VMEM((N_PAD // NS // 5, D_OUT), jnp.float32),
            pltpu.VMEM_SHARED((N_PAD, D_OUT), jnp.float32),
        ]
        + [pltpu.SemaphoreType.DMA] * (2 * GD)
    ),
)


ROW_BLK = 10000


def _prep_body(x_ref, w_ref, da_ref, db_ref, h0_ref, nrm_ref, inv_ref):
    deg = jnp.maximum(da_ref[...] + db_ref[...], 1.0)
    nrm = lax.rsqrt(deg)
    h0_ref[...] = jnp.dot(x_ref[...], w_ref[...],
                          preferred_element_type=jnp.float32) * nrm
    nrm_ref[...] = nrm
    inv_ref[...] = 1.0 / deg


def _mid_body(ga_ref, gb_ref, inv_ref, h1_ref):
    h1_ref[...] = (ga_ref[...] + gb_ref[...]) * inv_ref[...]


def _final_body(ga_ref, gb_ref, nrm_ref, b_ref, out_ref):
    out_ref[...] = (ga_ref[...] + gb_ref[...]) * nrm_ref[...] + b_ref[...]


def _row_spec(cols):
    return pl.BlockSpec((ROW_BLK, cols), lambda i: (i, 0))


def kernel(in_feat, edge_index, W, b):
    src3d = edge_index[0].astype(jnp.int32).reshape(NW, ROWS_PER_TILE, CHUNK)
    dst3d = edge_index[1].astype(jnp.int32).reshape(NW, ROWS_PER_TILE, CHUNK)

    deg_part = _deg_kernel(dst3d).reshape(NC, N_PAD)
    da = deg_part[0, :N_NODES].reshape(N_NODES, 1)
    db = deg_part[1, :N_NODES].reshape(N_NODES, 1)

    grid = (N_NODES // ROW_BLK,)
    h0, nrm, inv = pl.pallas_call(
        _prep_body,
        grid=grid,
        in_specs=[
            _row_spec(D_IN),
            pl.BlockSpec((D_IN, D_OUT), lambda i: (0, 0)),
            _row_spec(1),
            _row_spec(1),
        ],
        out_specs=[_row_spec(D_OUT), _row_spec(1), _row_spec(1)],
        out_shape=[
            jax.ShapeDtypeStruct((N_NODES, D_OUT), jnp.float32),
            jax.ShapeDtypeStruct((N_NODES, 1), jnp.float32),
            jax.ShapeDtypeStruct((N_NODES, 1), jnp.float32),
        ],
    )(in_feat, W, da, db)

    g1 = _hop_kernel(h0, src3d, dst3d)

    h1 = pl.pallas_call(
        _mid_body,
        grid=grid,
        in_specs=[_row_spec(D_OUT), _row_spec(D_OUT), _row_spec(1)],
        out_specs=_row_spec(D_OUT),
        out_shape=jax.ShapeDtypeStruct((N_NODES, D_OUT), jnp.float32),
    )(g1[0, :N_NODES], g1[1, :N_NODES], inv)

    g2 = _hop_kernel(h1, src3d, dst3d)

    out = pl.pallas_call(
        _final_body,
        grid=grid,
        in_specs=[
            _row_spec(D_OUT),
            _row_spec(D_OUT),
            _row_spec(1),
            pl.BlockSpec((1, D_OUT), lambda i: (0, 0)),
        ],
        out_specs=_row_spec(D_OUT),
        out_shape=jax.ShapeDtypeStruct((N_NODES, D_OUT), jnp.float32),
    )(g2[0, :N_NODES], g2[1, :N_NODES], nrm, b.reshape(1, D_OUT))

    return out


# speedup vs baseline: 1.4017x; 1.0081x over previous
import jax
import jax.numpy as jnp
from jax import lax
from jax.experimental import pallas as pl
from jax.experimental.pallas import tpu as pltpu
from jax.experimental.pallas import tpu_sc as plsc

N_NODES = 10000
N_EDGES = 320000
D_IN = 128
D_OUT = 64

NC = 2
NS = 16
NW = NC * NS
CHUNK = 80
ROWS_PER_TILE = N_EDGES // (NW * CHUNK)
N_PAD = 10240
NPT = N_PAD // NS

_mesh = plsc.VectorSubcoreMesh(
    core_axis_name="c", subcore_axis_name="s", num_cores=NC, num_subcores=NS)


def _deg_body(dst3d_hbm, out_hbm, dst_idx, ones_v, zero_v, shared_deg, sem,
              ssem):
    c = lax.axis_index("c")
    s = lax.axis_index("s")

    def fill(i, _):
        ones_v[pl.ds(i * 16, 16)] = jnp.ones((16,), jnp.float32)
        return 0
    lax.fori_loop(0, CHUNK // 16, fill, 0)

    def zfill(i, _):
        zero_v[pl.ds(i * 16, 16)] = jnp.zeros((16,), jnp.float32)
        return 0
    lax.fori_loop(0, NPT // 16, zfill, 0)

    pltpu.sync_copy(zero_v, shared_deg.at[pl.ds(s * NPT, NPT)])
    plsc.subcore_barrier()

    pltpu.sync_copy(dst3d_hbm.at[c * NS + s], dst_idx)

    def body(j, _):
        pltpu.async_copy(ones_v, shared_deg.at[dst_idx.at[j]], ssem, add=True)
        return 0
    lax.fori_loop(0, ROWS_PER_TILE, body, 0)

    def drain(j, _):
        pltpu.make_async_copy(ones_v, shared_deg.at[dst_idx.at[0]], ssem).wait()
        return 0
    lax.fori_loop(0, ROWS_PER_TILE, drain, 0)

    plsc.subcore_barrier()
    pltpu.sync_copy(shared_deg.at[pl.ds(s * NPT, NPT)],
                    out_hbm.at[pl.ds(c * N_PAD + s * NPT, NPT)])


_deg_kernel = pl.kernel(
    _deg_body,
    out_type=jax.ShapeDtypeStruct((NC * N_PAD,), jnp.float32),
    mesh=_mesh,
    scratch_types=[
        pltpu.VMEM((ROWS_PER_TILE, CHUNK), jnp.int32),
        pltpu.VMEM((CHUNK,), jnp.float32),
        pltpu.VMEM((NPT,), jnp.float32),
        pltpu.VMEM_SHARED((N_PAD,), jnp.float32),
        pltpu.SemaphoreType.DMA,
        pltpu.SemaphoreType.DMA,
    ],
)


GD = 6


def _hop_body(h_hbm, src3d_hbm, dst3d_hbm, out_hbm,
              src_idx, dst_idx, *rest):
    bufs = rest[:2 * GD]
    zrows_v = rest[2 * GD]
    shared_g = rest[2 * GD + 1]
    sems = rest[2 * GD + 2:]
    c = lax.axis_index("c")
    s = lax.axis_index("s")

    zchunk = NPT // 5

    def zfill(k, _):
        zrows_v[k // 4, pl.ds((k % 4) * 16, 16)] = jnp.zeros((16,), jnp.float32)
        return 0
    lax.fori_loop(0, zchunk * 4, zfill, 0)

    pltpu.sync_copy(src3d_hbm.at[c * NS + s], src_idx)
    pltpu.sync_copy(dst3d_hbm.at[c * NS + s], dst_idx)

    def zcopy(k, _):
        pltpu.sync_copy(zrows_v, shared_g.at[pl.ds(s * NPT + k * zchunk, zchunk)])
        return 0
    lax.fori_loop(0, 5, zcopy, 0)
    plsc.subcore_barrier()

    def gather(j, buf, sem):
        pltpu.async_copy(h_hbm.at[src_idx.at[j]], buf, sem)

    def gwait(buf, sem):
        pltpu.make_async_copy(h_hbm.at[src_idx.at[0]], buf, sem).wait()

    def scat(j, buf):
        pltpu.sync_copy(buf, shared_g.at[dst_idx.at[j]], add=True)

    for i in range(GD):
        gather(i, bufs[i], sems[i])

    def body(m, _):
        def run(cur, nxt):
            for i in range(GD):
                jn = (m + 1) * GD + i

                @pl.when(jn < ROWS_PER_TILE)
                def _():
                    gather(jn, bufs[nxt + i], sems[nxt + i])
            for i in range(GD):
                j = m * GD + i

                @pl.when(j < ROWS_PER_TILE)
                def _():
                    gwait(bufs[cur + i], sems[cur + i])
                    scat(j, bufs[cur + i])

        @pl.when(m % 2 == 0)
        def _():
            run(0, GD)

        @pl.when(m % 2 == 1)
        def _():
            run(GD, 0)
        return 0
    lax.fori_loop(0, (ROWS_PER_TILE + GD - 1) // GD, body, 0)

    plsc.subcore_barrier()
    pltpu.sync_copy(shared_g.at[pl.ds(s * NPT, NPT)],
                    out_hbm.at[c, pl.ds(s * NPT, NPT)])


_hop_kernel = pl.kernel(
    _hop_body,
    out_type=jax.ShapeDtypeStruct((NC, N_PAD, D_OUT), jnp.float32),
    mesh=_mesh,
    compiler_params=pltpu.CompilerParams(use_tc_tiling_on_sc=False),
    scratch_types=(
        [
            pltpu.VMEM((ROWS_PER_TILE, CHUNK), jnp.int32),
            pltpu.VMEM((ROWS_PER_TILE, CHUNK), jnp.int32),
        ]
        + [pltpu.VMEM((CHUNK, D_OUT), jnp.float32)] * (2 * GD)
        + [
            pltpu.VMEM((N_PAD // NS // 5, D_OUT), jnp.float32),
            pltpu.VMEM_SHARED((N_PAD, D_OUT), jnp.float32),
        ]
        + [pltpu.SemaphoreType.DMA] * (2 * GD)
    ),
)


ROW_BLK = 10000


def _prep_body(x_ref, w_ref, da_ref, db_ref, h0_ref, nrm_ref, inv_ref):
    deg = jnp.maximum(da_ref[...] + db_ref[...], 1.0)
    nrm = lax.rsqrt(deg)
    h0_ref[...] = jnp.dot(x_ref[...], w_ref[...],
                          preferred_element_type=jnp.float32) * nrm
    nrm_ref[...] = nrm
    inv_ref[...] = 1.0 / deg


def _mid_body(ga_ref, gb_ref, inv_ref, h1_ref):
    h1_ref[...] = (ga_ref[...] + gb_ref[...]) * inv_ref[...]


def _final_body(ga_ref, gb_ref, nrm_ref, b_ref, out_ref):
    out_ref[...] = (ga_ref[...] + gb_ref[...]) * nrm_ref[...] + b_ref[...]


def _row_spec(cols):
    return pl.BlockSpec((ROW_BLK, cols), lambda i: (i, 0))


def kernel(in_feat, edge_index, W, b):
    src3d = edge_index[0].astype(jnp.int32).reshape(NW, ROWS_PER_TILE, CHUNK)
    dst3d = edge_index[1].astype(jnp.int32).reshape(NW, ROWS_PER_TILE, CHUNK)

    deg_part = _deg_kernel(dst3d).reshape(NC, N_PAD)
    da = deg_part[0, :N_NODES].reshape(N_NODES, 1)
    db = deg_part[1, :N_NODES].reshape(N_NODES, 1)

    grid = (N_NODES // ROW_BLK,)
    h0, nrm, inv = pl.pallas_call(
        _prep_body,
        grid=grid,
        in_specs=[
            _row_spec(D_IN),
            pl.BlockSpec((D_IN, D_OUT), lambda i: (0, 0)),
            _row_spec(1),
            _row_spec(1),
        ],
        out_specs=[_row_spec(D_OUT), _row_spec(1), _row_spec(1)],
        out_shape=[
            jax.ShapeDtypeStruct((N_NODES, D_OUT), jnp.float32),
            jax.ShapeDtypeStruct((N_NODES, 1), jnp.float32),
            jax.ShapeDtypeStruct((N_NODES, 1), jnp.float32),
        ],
    )(in_feat, W, da, db)

    g1 = _hop_kernel(h0, src3d, dst3d)

    h1 = pl.pallas_call(
        _mid_body,
        grid=grid,
        in_specs=[_row_spec(D_OUT), _row_spec(D_OUT), _row_spec(1)],
        out_specs=_row_spec(D_OUT),
        out_shape=jax.ShapeDtypeStruct((N_NODES, D_OUT), jnp.float32),
    )(g1[0, :N_NODES], g1[1, :N_NODES], inv)

    g2 = _hop_kernel(h1, src3d, dst3d)

    out = pl.pallas_call(
        _final_body,
        grid=grid,
        in_specs=[
            _row_spec(D_OUT),
            _row_spec(D_OUT),
            _row_spec(1),
            pl.BlockSpec((1, D_OUT), lambda i: (0, 0)),
        ],
        out_specs=_row_spec(D_OUT),
        out_shape=jax.ShapeDtypeStruct((N_NODES, D_OUT), jnp.float32),
    )(g2[0, :N_NODES], g2[1, :N_NODES], nrm, b.reshape(1, D_OUT))

    return out


# speedup vs baseline: 1.4230x; 1.0152x over previous
import jax
import jax.numpy as jnp
from jax import lax
from jax.experimental import pallas as pl
from jax.experimental.pallas import tpu as pltpu
from jax.experimental.pallas import tpu_sc as plsc

N_NODES = 10000
N_EDGES = 320000
D_IN = 128
D_OUT = 64

NC = 2
NS = 16
NW = NC * NS
CHUNK = 80
ROWS_PER_TILE = N_EDGES // (NW * CHUNK)
N_PAD = 10240
NPT = N_PAD // NS

_mesh = plsc.VectorSubcoreMesh(
    core_axis_name="c", subcore_axis_name="s", num_cores=NC, num_subcores=NS)


def _deg_body(dst3d_hbm, out_hbm, dst_idx, ones_v, zero_v, shared_deg, sem,
              ssem):
    c = lax.axis_index("c")
    s = lax.axis_index("s")

    def fill(i, _):
        ones_v[pl.ds(i * 16, 16)] = jnp.ones((16,), jnp.float32)
        return 0
    lax.fori_loop(0, CHUNK // 16, fill, 0)

    def zfill(i, _):
        zero_v[pl.ds(i * 16, 16)] = jnp.zeros((16,), jnp.float32)
        return 0
    lax.fori_loop(0, NPT // 16, zfill, 0)

    pltpu.sync_copy(zero_v, shared_deg.at[pl.ds(s * NPT, NPT)])
    plsc.subcore_barrier()

    pltpu.sync_copy(dst3d_hbm.at[c * NS + s], dst_idx)

    def body(j, _):
        pltpu.async_copy(ones_v, shared_deg.at[dst_idx.at[j]], ssem, add=True)
        return 0
    lax.fori_loop(0, ROWS_PER_TILE, body, 0)

    def drain(j, _):
        pltpu.make_async_copy(ones_v, shared_deg.at[dst_idx.at[0]], ssem).wait()
        return 0
    lax.fori_loop(0, ROWS_PER_TILE, drain, 0)

    plsc.subcore_barrier()
    pltpu.sync_copy(shared_deg.at[pl.ds(s * NPT, NPT)],
                    out_hbm.at[pl.ds(c * N_PAD + s * NPT, NPT)])


_deg_kernel = pl.kernel(
    _deg_body,
    out_type=jax.ShapeDtypeStruct((NC * N_PAD,), jnp.float32),
    mesh=_mesh,
    scratch_types=[
        pltpu.VMEM((ROWS_PER_TILE, CHUNK), jnp.int32),
        pltpu.VMEM((CHUNK,), jnp.float32),
        pltpu.VMEM((NPT,), jnp.float32),
        pltpu.VMEM_SHARED((N_PAD,), jnp.float32),
        pltpu.SemaphoreType.DMA,
        pltpu.SemaphoreType.DMA,
    ],
)


GD = 6


def _hop_body(h_hbm, src3d_hbm, dst3d_hbm, out_hbm,
              src_idx, dst_idx, *rest):
    bufs = rest[:2 * GD]
    zrows_v = rest[2 * GD]
    shared_g = rest[2 * GD + 1]
    sems = rest[2 * GD + 2:4 * GD + 2]
    ssems = rest[4 * GD + 2:]
    c = lax.axis_index("c")
    s = lax.axis_index("s")

    zchunk = NPT // 5

    def zfill(k, _):
        zrows_v[k // 4, pl.ds((k % 4) * 16, 16)] = jnp.zeros((16,), jnp.float32)
        return 0
    lax.fori_loop(0, zchunk * 4, zfill, 0)

    pltpu.sync_copy(src3d_hbm.at[c * NS + s], src_idx)
    pltpu.sync_copy(dst3d_hbm.at[c * NS + s], dst_idx)

    def zcopy(k, _):
        pltpu.sync_copy(zrows_v, shared_g.at[pl.ds(s * NPT + k * zchunk, zchunk)])
        return 0
    lax.fori_loop(0, 5, zcopy, 0)
    plsc.subcore_barrier()

    def gather(j, buf, sem):
        pltpu.async_copy(h_hbm.at[src_idx.at[j]], buf, sem)

    def gwait(buf, sem):
        pltpu.make_async_copy(h_hbm.at[src_idx.at[0]], buf, sem).wait()

    def scat(j, buf, sem):
        pltpu.async_copy(buf, shared_g.at[dst_idx.at[j]], sem, add=True)

    def swait(buf, sem):
        pltpu.make_async_copy(buf, shared_g.at[dst_idx.at[0]], sem).wait()

    for i in range(GD):
        gather(i, bufs[i], sems[i])

    def body(m, _):
        def run(cur, nxt):
            for i in range(GD):
                jn = (m + 1) * GD + i

                @pl.when(jn < ROWS_PER_TILE)
                def _():
                    @pl.when(m > 0)
                    def _():
                        swait(bufs[nxt + i], ssems[nxt + i])
                    gather(jn, bufs[nxt + i], sems[nxt + i])
            for i in range(GD):
                j = m * GD + i

                @pl.when(j < ROWS_PER_TILE)
                def _():
                    gwait(bufs[cur + i], sems[cur + i])
                    scat(j, bufs[cur + i], ssems[cur + i])

        @pl.when(m % 2 == 0)
        def _():
            run(0, GD)

        @pl.when(m % 2 == 1)
        def _():
            run(GD, 0)
        return 0
    lax.fori_loop(0, (ROWS_PER_TILE + GD - 1) // GD, body, 0)

    for i in range(2 * GD):
        j = ROWS_PER_TILE - 1 - i

        @pl.when(j >= 0)
        def _():
            swait(bufs[0], ssems[(j // GD % 2) * GD + (j % GD)])

    plsc.subcore_barrier()
    pltpu.sync_copy(shared_g.at[pl.ds(s * NPT, NPT)],
                    out_hbm.at[c, pl.ds(s * NPT, NPT)])


_hop_kernel = pl.kernel(
    _hop_body,
    out_type=jax.ShapeDtypeStruct((NC, N_PAD, D_OUT), jnp.float32),
    mesh=_mesh,
    compiler_params=pltpu.CompilerParams(use_tc_tiling_on_sc=False),
    scratch_types=(
        [
            pltpu.VMEM((ROWS_PER_TILE, CHUNK), jnp.int32),
            pltpu.VMEM((ROWS_PER_TILE, CHUNK), jnp.int32),
        ]
        + [pltpu.VMEM((CHUNK, D_OUT), jnp.float32)] * (2 * GD)
        + [
            pltpu.VMEM((N_PAD // NS // 5, D_OUT), jnp.float32),
            pltpu.VMEM_SHARED((N_PAD, D_OUT), jnp.float32),
        ]
        + [pltpu.SemaphoreType.DMA] * (4 * GD)
    ),
)


ROW_BLK = 10000


def _prep_body(x_ref, w_ref, da_ref, db_ref, h0_ref, nrm_ref, inv_ref):
    deg = jnp.maximum(da_ref[...] + db_ref[...], 1.0)
    nrm = lax.rsqrt(deg)
    h0_ref[...] = jnp.dot(x_ref[...], w_ref[...],
                          preferred_element_type=jnp.float32) * nrm
    nrm_ref[...] = nrm
    inv_ref[...] = 1.0 / deg


def _mid_body(ga_ref, gb_ref, inv_ref, h1_ref):
    h1_ref[...] = (ga_ref[...] + gb_ref[...]) * inv_ref[...]


def _final_body(ga_ref, gb_ref, nrm_ref, b_ref, out_ref):
    out_ref[...] = (ga_ref[...] + gb_ref[...]) * nrm_ref[...] + b_ref[...]


def _row_spec(cols):
    return pl.BlockSpec((ROW_BLK, cols), lambda i: (i, 0))


def kernel(in_feat, edge_index, W, b):
    src3d = edge_index[0].astype(jnp.int32).reshape(NW, ROWS_PER_TILE, CHUNK)
    dst3d = edge_index[1].astype(jnp.int32).reshape(NW, ROWS_PER_TILE, CHUNK)

    deg_part = _deg_kernel(dst3d).reshape(NC, N_PAD)
    da = deg_part[0, :N_NODES].reshape(N_NODES, 1)
    db = deg_part[1, :N_NODES].reshape(N_NODES, 1)

    grid = (N_NODES // ROW_BLK,)
    h0, nrm, inv = pl.pallas_call(
        _prep_body,
        grid=grid,
        in_specs=[
            _row_spec(D_IN),
            pl.BlockSpec((D_IN, D_OUT), lambda i: (0, 0)),
            _row_spec(1),
            _row_spec(1),
        ],
        out_specs=[_row_spec(D_OUT), _row_spec(1), _row_spec(1)],
        out_shape=[
            jax.ShapeDtypeStruct((N_NODES, D_OUT), jnp.float32),
            jax.ShapeDtypeStruct((N_NODES, 1), jnp.float32),
            jax.ShapeDtypeStruct((N_NODES, 1), jnp.float32),
        ],
    )(in_feat, W, da, db)

    g1 = _hop_kernel(h0, src3d, dst3d)

    h1 = pl.pallas_call(
        _mid_body,
        grid=grid,
        in_specs=[_row_spec(D_OUT), _row_spec(D_OUT), _row_spec(1)],
        out_specs=_row_spec(D_OUT),
        out_shape=jax.ShapeDtypeStruct((N_NODES, D_OUT), jnp.float32),
    )(g1[0, :N_NODES], g1[1, :N_NODES], inv)

    g2 = _hop_kernel(h1, src3d, dst3d)

    out = pl.pallas_call(
        _final_body,
        grid=grid,
        in_specs=[
            _row_spec(D_OUT),
            _row_spec(D_OUT),
            _row_spec(1),
            pl.BlockSpec((1, D_OUT), lambda i: (0, 0)),
        ],
        out_specs=_row_spec(D_OUT),
        out_shape=jax.ShapeDtypeStruct((N_NODES, D_OUT), jnp.float32),
    )(g2[0, :N_NODES], g2[1, :N_NODES], nrm, b.reshape(1, D_OUT))

    return out


# speedup vs baseline: 1.4450x; 1.0155x over previous
import jax
import jax.numpy as jnp
from jax import lax
from jax.experimental import pallas as pl
from jax.experimental.pallas import tpu as pltpu
from jax.experimental.pallas import tpu_sc as plsc

N_NODES = 10000
N_EDGES = 320000
D_IN = 128
D_OUT = 64

NC = 2
NS = 16
NW = NC * NS
CHUNK = 80
ROWS_PER_TILE = N_EDGES // (NW * CHUNK)
N_PAD = 10240
NPT = N_PAD // NS

_mesh = plsc.VectorSubcoreMesh(
    core_axis_name="c", subcore_axis_name="s", num_cores=NC, num_subcores=NS)


def _deg_body(dst3d_hbm, out_hbm, dst_idx, ones_v, zero_v, shared_deg, sem,
              ssem):
    c = lax.axis_index("c")
    s = lax.axis_index("s")

    def fill(i, _):
        ones_v[pl.ds(i * 16, 16)] = jnp.ones((16,), jnp.float32)
        return 0
    lax.fori_loop(0, CHUNK // 16, fill, 0)

    def zfill(i, _):
        zero_v[pl.ds(i * 16, 16)] = jnp.zeros((16,), jnp.float32)
        return 0
    lax.fori_loop(0, NPT // 16, zfill, 0)

    pltpu.sync_copy(zero_v, shared_deg.at[pl.ds(s * NPT, NPT)])
    plsc.subcore_barrier()

    pltpu.sync_copy(dst3d_hbm.at[c * NS + s], dst_idx)

    def body(j, _):
        pltpu.async_copy(ones_v, shared_deg.at[dst_idx.at[j]], ssem, add=True)
        return 0
    lax.fori_loop(0, ROWS_PER_TILE, body, 0)

    def drain(j, _):
        pltpu.make_async_copy(ones_v, shared_deg.at[dst_idx.at[0]], ssem).wait()
        return 0
    lax.fori_loop(0, ROWS_PER_TILE, drain, 0)

    plsc.subcore_barrier()
    pltpu.sync_copy(shared_deg.at[pl.ds(s * NPT, NPT)],
                    out_hbm.at[pl.ds(c * N_PAD + s * NPT, NPT)])


_deg_kernel = pl.kernel(
    _deg_body,
    out_type=jax.ShapeDtypeStruct((NC * N_PAD,), jnp.float32),
    mesh=_mesh,
    scratch_types=[
        pltpu.VMEM((ROWS_PER_TILE, CHUNK), jnp.int32),
        pltpu.VMEM((CHUNK,), jnp.float32),
        pltpu.VMEM((NPT,), jnp.float32),
        pltpu.VMEM_SHARED((N_PAD,), jnp.float32),
        pltpu.SemaphoreType.DMA,
        pltpu.SemaphoreType.DMA,
    ],
)


GD = 6


def _hop_body(h_hbm, src3d_hbm, dst3d_hbm, out_hbm,
              src_idx, dst_idx, *rest):
    bufs = rest[:2 * GD]
    zrows_v = rest[2 * GD]
    shared_g = rest[2 * GD + 1]
    sems = rest[2 * GD + 2:4 * GD + 2]
    ssems = rest[4 * GD + 2:]
    c = lax.axis_index("c")
    s = lax.axis_index("s")

    zchunk = NPT // 5

    def gather(j, buf, sem):
        pltpu.async_copy(h_hbm.at[src_idx.at[j]], buf, sem)

    pltpu.sync_copy(src3d_hbm.at[c * NS + s], src_idx)
    pltpu.sync_copy(dst3d_hbm.at[c * NS + s], dst_idx)
    for i in range(GD):
        gather(i, bufs[i], sems[i])

    def zfill(k, _):
        zrows_v[k // 4, pl.ds((k % 4) * 16, 16)] = jnp.zeros((16,), jnp.float32)
        return 0
    lax.fori_loop(0, zchunk * 4, zfill, 0)

    def zcopy(k, _):
        pltpu.sync_copy(zrows_v, shared_g.at[pl.ds(s * NPT + k * zchunk, zchunk)])
        return 0
    lax.fori_loop(0, 5, zcopy, 0)
    plsc.subcore_barrier()

    def gwait(buf, sem):
        pltpu.make_async_copy(h_hbm.at[src_idx.at[0]], buf, sem).wait()

    def scat(j, buf, sem):
        pltpu.async_copy(buf, shared_g.at[dst_idx.at[j]], sem, add=True)

    def swait(buf, sem):
        pltpu.make_async_copy(buf, shared_g.at[dst_idx.at[0]], sem).wait()

    def body(m, _):
        def run(cur, nxt):
            for i in range(GD):
                jn = (m + 1) * GD + i

                @pl.when(jn < ROWS_PER_TILE)
                def _():
                    @pl.when(m > 0)
                    def _():
                        swait(bufs[nxt + i], ssems[nxt + i])
                    gather(jn, bufs[nxt + i], sems[nxt + i])
            for i in range(GD):
                j = m * GD + i

                @pl.when(j < ROWS_PER_TILE)
                def _():
                    gwait(bufs[cur + i], sems[cur + i])
                    scat(j, bufs[cur + i], ssems[cur + i])

        @pl.when(m % 2 == 0)
        def _():
            run(0, GD)

        @pl.when(m % 2 == 1)
        def _():
            run(GD, 0)
        return 0
    lax.fori_loop(0, (ROWS_PER_TILE + GD - 1) // GD, body, 0)

    for i in range(2 * GD):
        j = ROWS_PER_TILE - 1 - i

        @pl.when(j >= 0)
        def _():
            swait(bufs[0], ssems[(j // GD % 2) * GD + (j % GD)])

    plsc.subcore_barrier()
    pltpu.sync_copy(shared_g.at[pl.ds(s * NPT, NPT)],
                    out_hbm.at[c, pl.ds(s * NPT, NPT)])


_hop_kernel = pl.kernel(
    _hop_body,
    out_type=jax.ShapeDtypeStruct((NC, N_PAD, D_OUT), jnp.float32),
    mesh=_mesh,
    compiler_params=pltpu.CompilerParams(use_tc_tiling_on_sc=False),
    scratch_types=(
        [
            pltpu.VMEM((ROWS_PER_TILE, CHUNK), jnp.int32),
            pltpu.VMEM((ROWS_PER_TILE, CHUNK), jnp.int32),
        ]
        + [pltpu.VMEM((CHUNK, D_OUT), jnp.float32)] * (2 * GD)
        + [
            pltpu.VMEM((N_PAD // NS // 5, D_OUT), jnp.float32),
            pltpu.VMEM_SHARED((N_PAD, D_OUT), jnp.float32),
        ]
        + [pltpu.SemaphoreType.DMA] * (4 * GD)
    ),
)


ROW_BLK = 10000


def _prep_body(x_ref, w_ref, da_ref, db_ref, h0_ref, nrm_ref, inv_ref):
    deg = jnp.maximum(da_ref[...] + db_ref[...], 1.0)
    nrm = lax.rsqrt(deg)
    h0_ref[...] = jnp.dot(x_ref[...], w_ref[...],
                          preferred_element_type=jnp.float32) * nrm
    nrm_ref[...] = nrm
    inv_ref[...] = 1.0 / deg


def _mid_body(ga_ref, gb_ref, inv_ref, h1_ref):
    h1_ref[...] = (ga_ref[...] + gb_ref[...]) * inv_ref[...]


def _final_body(ga_ref, gb_ref, nrm_ref, b_ref, out_ref):
    out_ref[...] = (ga_ref[...] + gb_ref[...]) * nrm_ref[...] + b_ref[...]


def _row_spec(cols):
    return pl.BlockSpec((ROW_BLK, cols), lambda i: (i, 0))


def kernel(in_feat, edge_index, W, b):
    src3d = edge_index[0].astype(jnp.int32).reshape(NW, ROWS_PER_TILE, CHUNK)
    dst3d = edge_index[1].astype(jnp.int32).reshape(NW, ROWS_PER_TILE, CHUNK)

    deg_part = _deg_kernel(dst3d).reshape(NC, N_PAD)
    da = deg_part[0, :N_NODES].reshape(N_NODES, 1)
    db = deg_part[1, :N_NODES].reshape(N_NODES, 1)

    grid = (N_NODES // ROW_BLK,)
    h0, nrm, inv = pl.pallas_call(
        _prep_body,
        grid=grid,
        in_specs=[
            _row_spec(D_IN),
            pl.BlockSpec((D_IN, D_OUT), lambda i: (0, 0)),
            _row_spec(1),
            _row_spec(1),
        ],
        out_specs=[_row_spec(D_OUT), _row_spec(1), _row_spec(1)],
        out_shape=[
            jax.ShapeDtypeStruct((N_NODES, D_OUT), jnp.float32),
            jax.ShapeDtypeStruct((N_NODES, 1), jnp.float32),
            jax.ShapeDtypeStruct((N_NODES, 1), jnp.float32),
        ],
    )(in_feat, W, da, db)

    g1 = _hop_kernel(h0, src3d, dst3d)

    h1 = pl.pallas_call(
        _mid_body,
        grid=grid,
        in_specs=[_row_spec(D_OUT), _row_spec(D_OUT), _row_spec(1)],
        out_specs=_row_spec(D_OUT),
        out_shape=jax.ShapeDtypeStruct((N_NODES, D_OUT), jnp.float32),
    )(g1[0, :N_NODES], g1[1, :N_NODES], inv)

    g2 = _hop_kernel(h1, src3d, dst3d)

    out = pl.pallas_call(
        _final_body,
        grid=grid,
        in_specs=[
            _row_spec(D_OUT),
            _row_spec(D_OUT),
            _row_spec(1),
            pl.BlockSpec((1, D_OUT), lambda i: (0, 0)),
        ],
        out_specs=_row_spec(D_OUT),
        out_shape=jax.ShapeDtypeStruct((N_NODES, D_OUT), jnp.float32),
    )(g2[0, :N_NODES], g2[1, :N_NODES], nrm, b.reshape(1, D_OUT))

    return out
